# Initial kernel scaffold; baseline (speedup 1.0000x reference)
import jax
import jax.numpy as jnp
from jax.experimental import pallas as pl


def kernel(indices, table):
    raise NotImplementedError("write your pallas kernel here")


# speedup vs baseline: 1.4361x; 1.4361x over previous
import functools

import jax
import jax.numpy as jnp
from jax import lax
from jax.experimental import pallas as pl
from jax.experimental.pallas import tpu as pltpu
from jax.experimental.pallas import tpu_sc as plsc

NC = 2
NS = 16
NW = NC * NS

CHUNK = 128


def _make_kernel(n_rows, embed_dim, n_chunks):
    mesh = plsc.VectorSubcoreMesh(core_axis_name="c", subcore_axis_name="s")
    rows_per_worker = n_chunks * CHUNK

    @functools.partial(
        pl.kernel,
        out_type=jax.ShapeDtypeStruct((n_rows, embed_dim), jnp.float32),
        mesh=mesh,
        scratch_types=[
            pltpu.VMEM((n_chunks, CHUNK), jnp.int32),
            pltpu.VMEM((CHUNK, embed_dim), jnp.float32),
            pltpu.SemaphoreType.DMA,
        ],
        compiler_params=pltpu.CompilerParams(use_tc_tiling_on_sc=False),
    )
    def gather_kernel(table_hbm, idx_hbm, out_hbm, idx_v, rows_v, sem):
        wid = lax.axis_index("s") * NC + lax.axis_index("c")
        base = wid * rows_per_worker
        pltpu.sync_copy(idx_hbm.at[wid], idx_v)

        def body(g, carry):
            pltpu.async_copy(table_hbm.at[idx_v.at[g]], rows_v, sem).wait()
            pltpu.sync_copy(
                rows_v, out_hbm.at[pl.ds(base + g * CHUNK, CHUNK)]
            )
            return carry

        lax.fori_loop(0, n_chunks, body, 0)

    return gather_kernel


def kernel(indices, table):
    batch, n_fields = indices.shape
    vocab, embed_dim = table.shape
    n_rows = batch * n_fields
    assert n_rows % (NW * CHUNK) == 0
    n_chunks = n_rows // (NW * CHUNK)

    idx3 = indices.reshape(NW, n_chunks, CHUNK)
    out_flat = _make_kernel(n_rows, embed_dim, n_chunks)(table, idx3)
    return out_flat.reshape(batch, n_fields, embed_dim)


# speedup vs baseline: 1.5760x; 1.0975x over previous
import functools

import jax
import jax.numpy as jnp
from jax import lax
from jax.experimental import pallas as pl
from jax.experimental.pallas import tpu as pltpu
from jax.experimental.pallas import tpu_sc as plsc

NC = 2
NS = 16
NW = NC * NS

CHUNK = 128
G = 8


def _make_kernel(n_rows, embed_dim, n_chunks):
    mesh = plsc.VectorSubcoreMesh(core_axis_name="c", subcore_axis_name="s")
    rows_per_worker = n_chunks * CHUNK
    n_groups = n_chunks // G
    group_rows = G * CHUNK

    @functools.partial(
        pl.kernel,
        out_type=jax.ShapeDtypeStruct((n_rows, embed_dim), jnp.float32),
        mesh=mesh,
        scratch_types=[
            pltpu.VMEM((n_chunks, CHUNK), jnp.int32),
            pltpu.VMEM((2, group_rows, embed_dim), jnp.float32),
            pltpu.SemaphoreType.DMA,
            pltpu.SemaphoreType.DMA,
            pltpu.SemaphoreType.DMA,
            pltpu.SemaphoreType.DMA,
        ],
        compiler_params=pltpu.CompilerParams(use_tc_tiling_on_sc=False),
    )
    def gather_kernel(table_hbm, idx_hbm, out_hbm, idx_v, rows_v,
                      sem_g0, sem_g1, sem_w0, sem_w1):
        wid = lax.axis_index("s") * NC + lax.axis_index("c")
        base = wid * rows_per_worker
        sem_g = (sem_g0, sem_g1)
        sem_w = (sem_w0, sem_w1)

        pltpu.sync_copy(idx_hbm.at[wid], idx_v)

        def gath(g, parity, j):
            return pltpu.make_async_copy(
                table_hbm.at[idx_v.at[g * G + j]],
                rows_v.at[parity, pl.ds(j * CHUNK, CHUNK)],
                sem_g[parity],
            )

        def writ(g, parity):
            return pltpu.make_async_copy(
                rows_v.at[parity],
                out_hbm.at[pl.ds(base + g * group_rows, group_rows)],
                sem_w[parity],
            )

        def fire(g, parity):
            for j in range(G):
                gath(g, parity, j).start()

        def step(g, parity, fire_ahead):
            for j in range(G):
                gath(g, parity, j).wait()
            writ(g, parity).start()
            if fire_ahead:
                writ(g, parity).wait()
                fire(g + 2, parity)

        fire(0, 0)
        fire(1, 1)

        n_reg = n_groups - 3
        n_reg -= n_reg % 2

        def body(i, carry):
            g = i * 2
            step(g, 0, True)
            step(g + 1, 1, True)
            return carry

        lax.fori_loop(0, n_reg // 2, body, 0)

        for g in range(n_reg, n_groups):
            step(g, g % 2, g + 2 < n_groups)
        for g in (n_groups - 2, n_groups - 1):
            writ(g, g % 2).wait()

    return gather_kernel


def kernel(indices, table):
    batch, n_fields = indices.shape
    vocab, embed_dim = table.shape
    n_rows = batch * n_fields
    assert n_rows % (NW * CHUNK) == 0
    n_chunks = n_rows // (NW * CHUNK)
    assert n_chunks % G == 0

    idx3 = indices.reshape(NW, n_chunks, CHUNK)
    out_flat = _make_kernel(n_rows, embed_dim, n_chunks)(table, idx3)
    return out_flat.reshape(batch, n_fields, embed_dim)
